# baseline (speedup 1.0000x reference)
import functools

import jax
import jax.numpy as jnp
from jax import lax
from jax.experimental import pallas as pl
from jax.experimental.pallas import tpu as pltpu
from jax.experimental.pallas import tpu_sc as plsc

NX = NY = NZ = 256
NXYZ = NX * NY * NZ
N_POINTS = 524288
NW = 32
PTS_PER_W = N_POINTS // NW
C = 128
G = C // 16
NCHUNK = PTS_PER_W // C
D = 8
NROWS2 = 2 * NXYZ // D

_CORNER_COLS = ((1, 1), (1, 2), (2, 1), (2, 2))
_EXT_COLS = ((0, 1), (0, 2), (3, 1), (3, 2), (1, 0), (2, 0), (1, 3), (2, 3))
_COLS = _CORNER_COLS + _EXT_COLS
NCOL = len(_COLS)
NSTREAM = 2 * NCOL
_D2_CORNERS = ((1, 1, 1), (2, 1, 1), (1, 2, 1), (1, 1, 2),
               (2, 1, 2), (1, 2, 2), (2, 2, 1), (2, 2, 2))


def _make_sc_call():
    mesh = plsc.VectorSubcoreMesh(core_axis_name="c", subcore_axis_name="s")

    @functools.partial(
        pl.kernel,
        out_type=jax.ShapeDtypeStruct((N_POINTS,), jnp.float32),
        mesh=mesh,
        compiler_params=pltpu.CompilerParams(needs_layout_passes=False,
                                             use_tc_tiling_on_sc=False),
        scratch_types=[
            pltpu.VMEM((384,), jnp.float32),
            pltpu.VMEM((32,), jnp.float32),
            pltpu.VMEM((PTS_PER_W,), jnp.float32),
            pltpu.VMEM((PTS_PER_W,), jnp.float32),
            pltpu.VMEM((PTS_PER_W,), jnp.float32),
            pltpu.VMEM((PTS_PER_W,), jnp.float32),
            pltpu.VMEM((C,), jnp.float32),
            pltpu.VMEM((C,), jnp.float32),
            pltpu.VMEM((NSTREAM, C), jnp.int32),
            pltpu.VMEM((NSTREAM, C), jnp.int32),
            pltpu.VMEM((NSTREAM, C, D), jnp.float32),
            pltpu.VMEM((NSTREAM, C, D), jnp.float32),
            pltpu.VMEM((8, C), jnp.int32),
            pltpu.VMEM((8, C), jnp.int32),
            pltpu.SemaphoreType.DMA,
            pltpu.SemaphoreType.DMA,
            pltpu.SemaphoreType.DMA,
        ],
    )
    def body(xs_hbm, ys_hbm, zs_hbm, phi_hbm, tb8_hbm, xg_hbm, cst_hbm,
             out_hbm, coord_v, cst_v, xv, yv, zv, pv, res_a, res_b,
             idx_a, idx_b, val_a, val_b, aux_a, aux_b,
             sem_a, sem_b, sem_o):
        wid = lax.axis_index("s") * 2 + lax.axis_index("c")
        tbase = wid * PTS_PER_W
        pltpu.sync_copy(xg_hbm, coord_v)
        pltpu.sync_copy(cst_hbm, cst_v)
        pltpu.sync_copy(xs_hbm.at[pl.ds(tbase, PTS_PER_W)], xv)
        pltpu.sync_copy(ys_hbm.at[pl.ds(tbase, PTS_PER_W)], yv)
        pltpu.sync_copy(zs_hbm.at[pl.ds(tbase, PTS_PER_W)], zv)
        pltpu.sync_copy(phi_hbm.at[pl.ds(tbase, PTS_PER_W)], pv)
        xg0 = cst_v[pl.ds(0, 16)]
        dxv = cst_v[pl.ds(16, 16)]
        iota16 = lax.iota(jnp.int32, 16)

        def axis_calc(p):
            t = (p - xg0) / dxv
            i = t.astype(jnp.int32)
            i = jnp.clip(i, 2, 256)
            ci = plsc.load_gather(coord_v, [i])
            ci1 = plsc.load_gather(coord_v, [i + 1])
            fd = (p - ci) / (ci1 - ci)
            a = i - 1
            return fd, (a - 1, a, jnp.minimum(a + 1, NX - 1),
                        jnp.minimum(a + 2, NX - 1))

        def fire(t, idx_v, aux_v, sem):
            val_v = val_of[id(idx_v)]

            def grp(g, carry):
                sl = pl.ds(g * 16, 16)
                psl = pl.ds(t * C + g * 16, 16)
                fx, xi = axis_calc(xv[psl])
                fy, yi = axis_calc(yv[psl])
                fz, zi = axis_calc(zv[psl])
                off8 = jnp.where(pv[psl] >= 0.0,
                                 jnp.int32(NXYZ // D), jnp.int32(0))
                aux_v[0, sl] = plsc.bitcast(fx, jnp.int32)
                aux_v[1, sl] = plsc.bitcast(fy, jnp.int32)
                aux_v[2, sl] = plsc.bitcast(fz, jnp.int32)
                zb_c = zi[0] & ~7
                zb_e = zi[1] & ~7
                aux_v[3, sl] = zi[0] - zb_c
                aux_v[4, sl] = zi[1] - zb_e
                aux_v[5, sl] = zi[2] - zb_e
                aux_v[6, sl] = zi[3] - zb_c
                aux_v[7, sl] = zi[2] - zb_c
                rz_c = zb_c >> 3
                rz_e = zb_e >> 3
                for j, (c, d) in enumerate(_COLS):
                    colb8 = (xi[c] * NY + yi[d]) * (NZ // D) + off8
                    r0 = colb8 + (rz_c if j < 4 else rz_e)
                    idx_v[2 * j, sl] = r0
                    idx_v[2 * j + 1, sl] = r0 + 1
                return carry
            lax.fori_loop(0, G, grp, 0)
            for s in range(NSTREAM):
                pltpu.async_copy(tb8_hbm.at[idx_v.at[s]], val_v.at[s], sem)

        val_of = {id(idx_a): val_a, id(idx_b): val_b}

        def drain(idx_v, sem):
            val_v = val_of[id(idx_v)]
            for s in range(NSTREAM):
                pltpu.make_async_copy(tb8_hbm.at[idx_v.at[s]],
                                      val_v.at[s], sem).wait()

        def math(t, idx_v, aux_v, res_v):
            val_v = val_of[id(idx_v)]

            def grp(g, carry):
                sl = pl.ds(g * 16, 16)
                pt = iota16 + g * 16
                fx = plsc.bitcast(aux_v[0, sl], jnp.float32)
                fy = plsc.bitcast(aux_v[1, sl], jnp.float32)
                fz = plsc.bitcast(aux_v[2, sl], jnp.float32)
                p_c0 = aux_v[3, sl]
                p_e1 = aux_v[4, sl]
                p_e2 = aux_v[5, sl]
                p_c3 = aux_v[6, sl]
                p_c2 = aux_v[7, sl]
                p_c1 = p_c0 + 1
                v = {}
                for j, (c, d) in enumerate(_COLS):
                    if j < 4:
                        poss = ((0, p_c0), (1, p_c1), (2, p_c2), (3, p_c3))
                    else:
                        poss = ((1, p_e1), (2, p_e2))
                    for e, pos in poss:
                        row = 2 * j + (pos >> 3)
                        v[(c, d, e)] = plsc.load_gather(
                            val_v, [row, pt, pos & 7])
                c00 = v[1, 1, 1] * (1.0 - fx) + v[2, 1, 1] * fx
                c01 = v[1, 1, 2] * (1.0 - fx) + v[2, 1, 2] * fx
                c10 = v[1, 2, 1] * (1.0 - fx) + v[2, 2, 1] * fx
                c11 = v[1, 2, 2] * (1.0 - fx) + v[2, 2, 2] * fx
                c0 = c00 * (1.0 - fy) + c10 * fy
                c1 = c01 * (1.0 - fy) + c11 * fy
                cval = c0 * (1.0 - fz) + c1 * fz
                mdx = mdy = mdz = None
                for (c, d, e) in _D2_CORNERS:
                    d2x = jnp.abs(v[c + 1, d, e] - 2.0 * v[c, d, e]
                                  + v[c - 1, d, e])
                    d2y = jnp.abs(v[c, d + 1, e] - 2.0 * v[c, d, e]
                                  + v[c, d - 1, e])
                    d2z = jnp.abs(v[c, d, e + 1] - 2.0 * v[c, d, e]
                                  + v[c, d, e - 1])
                    mdx = d2x if mdx is None else jnp.minimum(mdx, d2x)
                    mdy = d2y if mdy is None else jnp.minimum(mdy, d2y)
                    mdz = d2z if mdz is None else jnp.minimum(mdz, d2z)
                cval = (cval
                        - mdx * 0.5 * fx * (1.0 - fx)
                        - mdy * 0.5 * fy * (1.0 - fy)
                        - mdz * 0.5 * fz * (1.0 - fz))
                res_v[sl] = cval
                return carry
            lax.fori_loop(0, G, grp, 0)

        def out_fire(t, res_v):
            pltpu.async_copy(res_v, out_hbm.at[pl.ds(tbase + t * C, C)],
                             sem_o)

        def out_drain(t, res_v):
            pltpu.make_async_copy(
                res_v, out_hbm.at[pl.ds(tbase + t * C, C)], sem_o).wait()

        fire(0, idx_a, aux_a, sem_a)

        def pipe(k, carry):
            te = 2 * k
            fire(te + 1, idx_b, aux_b, sem_b)
            drain(idx_a, sem_a)

            @pl.when(k > 0)
            def _():
                out_drain(te - 2, res_a)
            math(te, idx_a, aux_a, res_a)
            out_fire(te, res_a)

            @pl.when(k < NCHUNK // 2 - 1)
            def _():
                fire(te + 2, idx_a, aux_a, sem_a)

            drain(idx_b, sem_b)

            @pl.when(k > 0)
            def _():
                out_drain(te - 1, res_b)
            math(te + 1, idx_b, aux_b, res_b)
            out_fire(te + 1, res_b)
            return carry

        lax.fori_loop(0, NCHUNK // 2, pipe, 0)
        out_drain(NCHUNK - 2, res_a)
        out_drain(NCHUNK - 1, res_b)

    return body


_SC_CALL = _make_sc_call()


def kernel(r, phi_r, trainables_m, trainables_p):
    xc = jnp.linspace(-1.0, 1.0, NX, dtype=jnp.float32)
    dx = xc[1] - xc[0]
    xg = jnp.concatenate([xc[:1] - dx, xc, xc[-1:] + dx])
    xg = jnp.pad(xg, (0, 384 - NX - 2))
    tb8 = jnp.concatenate(
        [trainables_m.reshape(-1), trainables_p.reshape(-1)]).reshape(-1, D)
    cst = jnp.concatenate([jnp.full((16,), xg[0]), jnp.full((16,), dx)])
    rt = r.T
    return _SC_CALL(rt[0], rt[1], rt[2], phi_r, tb8, xg, cst)


# speedup vs baseline: 5.4568x; 5.4568x over previous
import functools

import jax
import jax.numpy as jnp
from jax import lax
from jax.experimental import pallas as pl
from jax.experimental.pallas import tpu as pltpu
from jax.experimental.pallas import tpu_sc as plsc

NX = NY = NZ = 256
NXYZ = NX * NY * NZ
N_POINTS = 524288
NW = 32
PTS_PER_W = N_POINTS // NW
C = 128
G = C // 16
NCHUNK = PTS_PER_W // C
NSLOT = 32

_SLOTS = (
    [(c, d, e) for (c, d) in ((1, 1), (1, 2), (2, 1), (2, 2)) for e in range(4)]
    + [(c, d, e) for c in (0, 3) for d in (1, 2) for e in (1, 2)]
    + [(c, d, e) for c in (1, 2) for d in (0, 3) for e in (1, 2)]
)
assert len(_SLOTS) == NSLOT
_D2_CORNERS = ((1, 1, 1), (2, 1, 1), (1, 2, 1), (1, 1, 2),
               (2, 1, 2), (1, 2, 2), (2, 2, 1), (2, 2, 2))


def _make_sc_call():
    mesh = plsc.VectorSubcoreMesh(core_axis_name="c", subcore_axis_name="s")

    @functools.partial(
        pl.kernel,
        out_type=jax.ShapeDtypeStruct((N_POINTS,), jnp.float32),
        mesh=mesh,
        compiler_params=pltpu.CompilerParams(needs_layout_passes=False,
                                             use_tc_tiling_on_sc=False),
        scratch_types=[
            pltpu.VMEM((384,), jnp.float32),
            pltpu.VMEM((32,), jnp.float32),
            pltpu.VMEM((PTS_PER_W,), jnp.float32),
            pltpu.VMEM((PTS_PER_W,), jnp.float32),
            pltpu.VMEM((PTS_PER_W,), jnp.float32),
            pltpu.VMEM((PTS_PER_W,), jnp.float32),
            pltpu.VMEM((PTS_PER_W,), jnp.float32),
            pltpu.VMEM((NSLOT, C), jnp.int32),
            pltpu.VMEM((NSLOT, C), jnp.int32),
            pltpu.VMEM((NSLOT, C), jnp.float32),
            pltpu.VMEM((NSLOT, C), jnp.float32),
            pltpu.VMEM((4, C), jnp.float32),
            pltpu.VMEM((4, C), jnp.float32),
            pltpu.SemaphoreType.DMA,
            pltpu.SemaphoreType.DMA,
        ],
    )
    def body(xs_hbm, ys_hbm, zs_hbm, phi_hbm, tb_hbm, xg_hbm, cst_hbm,
             out_hbm, coord_v, cst_v, xv, yv, zv, pv, rv,
             idx_a, idx_b, val_a, val_b, frac_a, frac_b, sem_a, sem_b):
        wid = lax.axis_index("s") * 2 + lax.axis_index("c")
        tbase = wid * PTS_PER_W
        pltpu.sync_copy(xg_hbm, coord_v)
        pltpu.sync_copy(cst_hbm, cst_v)
        pltpu.sync_copy(xs_hbm.at[pl.ds(tbase, PTS_PER_W)], xv)
        pltpu.sync_copy(ys_hbm.at[pl.ds(tbase, PTS_PER_W)], yv)
        pltpu.sync_copy(zs_hbm.at[pl.ds(tbase, PTS_PER_W)], zv)
        pltpu.sync_copy(phi_hbm.at[pl.ds(tbase, PTS_PER_W)], pv)
        xg0 = cst_v[pl.ds(0, 16)]
        dxv = cst_v[pl.ds(16, 16)]

        def axis_calc(p):
            t = (p - xg0) / dxv
            i = t.astype(jnp.int32)
            i = jnp.clip(i, 2, 256)
            ci = plsc.load_gather(coord_v, [i])
            ci1 = plsc.load_gather(coord_v, [i + 1])
            fd = (p - ci) / (ci1 - ci)
            a = i - 1
            return fd, (a - 1, a, jnp.minimum(a + 1, NX - 1),
                        jnp.minimum(a + 2, NX - 1))

        def fire(t, idx_v, frac_v, sem):
            def grp(g, carry):
                sl = pl.ds(g * 16, 16)
                psl = pl.ds(t * C + g * 16, 16)
                fx, xi = axis_calc(xv[psl])
                fy, yi = axis_calc(yv[psl])
                fz, zi = axis_calc(zv[psl])
                off = jnp.where(pv[psl] >= 0.0, jnp.int32(NXYZ), jnp.int32(0))
                frac_v[0, sl] = fx
                frac_v[1, sl] = fy
                frac_v[2, sl] = fz
                cols = {}
                for (c, d, _e) in _SLOTS:
                    if (c, d) not in cols:
                        cols[(c, d)] = (xi[c] * NY + yi[d]) * NZ + off
                for s, (c, d, e) in enumerate(_SLOTS):
                    idx_v[s, sl] = cols[(c, d)] + zi[e]
                return carry
            lax.fori_loop(0, G, grp, 0)
            for s in range(NSLOT):
                pltpu.async_copy(tb_hbm.at[idx_v.at[s]], val_v_of[id(idx_v)].at[s], sem)

        val_v_of = {id(idx_a): val_a, id(idx_b): val_b}

        def drain(idx_v, sem):
            val_v = val_v_of[id(idx_v)]
            for s in range(NSLOT):
                pltpu.make_async_copy(tb_hbm.at[idx_v.at[s]],
                                      val_v.at[s], sem).wait()

        def math(t, idx_v, frac_v):
            val_v = val_v_of[id(idx_v)]

            def grp(g, carry):
                sl = pl.ds(g * 16, 16)
                fx = frac_v[0, sl]
                fy = frac_v[1, sl]
                fz = frac_v[2, sl]
                v = {cde: val_v[s, sl] for s, cde in enumerate(_SLOTS)}
                c00 = v[1, 1, 1] * (1.0 - fx) + v[2, 1, 1] * fx
                c01 = v[1, 1, 2] * (1.0 - fx) + v[2, 1, 2] * fx
                c10 = v[1, 2, 1] * (1.0 - fx) + v[2, 2, 1] * fx
                c11 = v[1, 2, 2] * (1.0 - fx) + v[2, 2, 2] * fx
                c0 = c00 * (1.0 - fy) + c10 * fy
                c1 = c01 * (1.0 - fy) + c11 * fy
                cval = c0 * (1.0 - fz) + c1 * fz
                mdx = mdy = mdz = None
                for (c, d, e) in _D2_CORNERS:
                    d2x = jnp.abs(v[c + 1, d, e] - 2.0 * v[c, d, e]
                                  + v[c - 1, d, e])
                    d2y = jnp.abs(v[c, d + 1, e] - 2.0 * v[c, d, e]
                                  + v[c, d - 1, e])
                    d2z = jnp.abs(v[c, d, e + 1] - 2.0 * v[c, d, e]
                                  + v[c, d, e - 1])
                    mdx = d2x if mdx is None else jnp.minimum(mdx, d2x)
                    mdy = d2y if mdy is None else jnp.minimum(mdy, d2y)
                    mdz = d2z if mdz is None else jnp.minimum(mdz, d2z)
                cval = (cval
                        - mdx * 0.5 * fx * (1.0 - fx)
                        - mdy * 0.5 * fy * (1.0 - fy)
                        - mdz * 0.5 * fz * (1.0 - fz))
                rv[pl.ds(t * C + g * 16, 16)] = cval
                return carry
            lax.fori_loop(0, G, grp, 0)

        fire(0, idx_a, frac_a, sem_a)

        def pipe(k, carry):
            te = 2 * k
            fire(te + 1, idx_b, frac_b, sem_b)
            drain(idx_a, sem_a)
            math(te, idx_a, frac_a)

            @pl.when(k < NCHUNK // 2 - 1)
            def _():
                fire(te + 2, idx_a, frac_a, sem_a)

            drain(idx_b, sem_b)
            math(te + 1, idx_b, frac_b)
            return carry

        lax.fori_loop(0, NCHUNK // 2, pipe, 0)
        pltpu.sync_copy(rv, out_hbm.at[pl.ds(tbase, PTS_PER_W)])

    return body


_SC_CALL = _make_sc_call()


def kernel(r, phi_r, trainables_m, trainables_p):
    xc = jnp.linspace(-1.0, 1.0, NX, dtype=jnp.float32)
    dx = xc[1] - xc[0]
    xg = jnp.concatenate([xc[:1] - dx, xc, xc[-1:] + dx])
    xg = jnp.pad(xg, (0, 384 - NX - 2))
    tb = jnp.concatenate([trainables_m.reshape(-1), trainables_p.reshape(-1)])
    cst = jnp.concatenate([jnp.full((16,), xg[0]), jnp.full((16,), dx)])
    rt = r.T
    return _SC_CALL(rt[0], rt[1], rt[2], phi_r, tb, xg, cst)
